# Initial kernel scaffold; baseline (speedup 1.0000x reference)
import jax
import jax.numpy as jnp
from jax.experimental import pallas as pl


def kernel(cls_logits, bbox_reg, centerness, anchors):
    raise NotImplementedError("write your pallas kernel here")


# speedup vs baseline: 1.5170x; 1.5170x over previous
import jax
import jax.numpy as jnp
from jax import lax
from jax.experimental import pallas as pl
from jax.experimental.pallas import tpu as pltpu

_N = 20000
_C = 80
_K = 1000
_KP = 1024
_CONF = 0.05
_IOU = 0.5
_OFF = 4096.0


def _scores_body(cls_ref, ctr_ref, out_ref):
    cs = jax.nn.sigmoid(ctr_ref[...])
    s = jax.nn.sigmoid(cls_ref[...])
    sc = s * cs
    out_ref[...] = jnp.where(sc > _CONF, sc, 0.0)


def _nms_body(ts_row_ref, ts_col_ref, ti_col_ref, ti_smem, bbox_ref,
              out_ref, boxes_ref, supu_ref, keep_ref):
    def gb(j, carry):
        r = ti_smem[j] // _C
        boxes_ref[pl.ds(j, 1), :] = bbox_ref[pl.ds(r, 1), :]
        return carry

    lax.fori_loop(0, _KP, gb, 0)

    lab_col = (ti_col_ref[...] % _C).astype(jnp.float32)
    offb = boxes_ref[...] + lab_col * _OFF

    ii = lax.broadcasted_iota(jnp.int32, (_KP, _KP), 0)
    jj = lax.broadcasted_iota(jnp.int32, (_KP, _KP), 1)
    ident = jnp.where(ii == jj, 1.0, 0.0)
    offr = lax.dot_general(offb, ident, (((0,), (0,)), ((), ())),
                           preferred_element_type=jnp.float32)

    x1c, y1c, x2c, y2c = (offb[:, k:k + 1] for k in range(4))
    x1r, y1r, x2r, y2r = (offr[k:k + 1, :] for k in range(4))
    area_c = jnp.maximum(x2c - x1c, 0.0) * jnp.maximum(y2c - y1c, 0.0)
    area_r = jnp.maximum(x2r - x1r, 0.0) * jnp.maximum(y2r - y1r, 0.0)
    ltx = jnp.maximum(x1c, x1r)
    lty = jnp.maximum(y1c, y1r)
    rbx = jnp.minimum(x2c, x2r)
    rby = jnp.minimum(y2c, y2r)
    inter = jnp.maximum(rbx - ltx, 0.0) * jnp.maximum(rby - lty, 0.0)
    union = area_c + area_r - inter
    iou = inter / jnp.maximum(union, 1e-9)
    supu_ref[...] = jnp.where((iou > _IOU) & (jj > ii), 1.0, 0.0)

    keep_ref[...] = jnp.where(ts_row_ref[...] > _CONF, 1.0, 0.0)
    lane = lax.broadcasted_iota(jnp.int32, (1, _KP), 1)

    def body(i, carry):
        row = supu_ref[pl.ds(i, 1), :]
        kv = keep_ref[...]
        ki = jnp.sum(jnp.where(lane == i, kv, 0.0))
        keep_ref[...] = kv * (1.0 - row * ki)
        return carry

    lax.fori_loop(0, _K, body, 0)

    keep_col = lax.dot_general(ident, keep_ref[...], (((1,), (1,)), ((), ())),
                               preferred_element_type=jnp.float32)

    out_ref[:, 0:4] = boxes_ref[0:_K, :]
    out_ref[:, 4:5] = ts_col_ref[0:_K, :]
    out_ref[:, 5:6] = keep_col[0:_K, :]


def kernel(cls_logits, bbox_reg, centerness, anchors):
    del anchors
    f32 = jnp.float32
    scores = pl.pallas_call(
        _scores_body,
        out_shape=jax.ShapeDtypeStruct((_N, _C), f32),
    )(cls_logits, centerness.reshape(_N, 1))

    top_s, top_i = lax.top_k(scores.reshape(-1), _K)
    pad = _KP - _K
    ts_p = jnp.concatenate([top_s, jnp.full((pad,), -1.0, f32)])
    ti_p = jnp.concatenate([top_i, jnp.zeros((pad,), jnp.int32)])

    out = pl.pallas_call(
        _nms_body,
        out_shape=jax.ShapeDtypeStruct((_K, 6), f32),
        in_specs=[
            pl.BlockSpec(memory_space=pltpu.VMEM),
            pl.BlockSpec(memory_space=pltpu.VMEM),
            pl.BlockSpec(memory_space=pltpu.VMEM),
            pl.BlockSpec(memory_space=pltpu.SMEM),
            pl.BlockSpec(memory_space=pltpu.VMEM),
        ],
        scratch_shapes=[
            pltpu.VMEM((_KP, 4), f32),
            pltpu.VMEM((_KP, _KP), f32),
            pltpu.VMEM((1, _KP), f32),
        ],
    )(ts_p.reshape(1, _KP), ts_p.reshape(_KP, 1), ti_p.reshape(_KP, 1),
      ti_p, bbox_reg)
    return out


# speedup vs baseline: 11.2428x; 7.4113x over previous
import functools

import jax
import jax.numpy as jnp
from jax import lax
from jax.experimental import pallas as pl
from jax.experimental.pallas import tpu as pltpu
from jax.experimental.pallas import tpu_sc as plsc

_N = 20000
_C = 80
_K = 1000
_KP = 1024
_CONF = 0.05
_IOU = 0.5
_OFF = 4096.0

_NW = 32
_SEG = _N * _C // _NW
_CAP = 256
_REFINE = 3


def _scores_body(cls_ref, ctr_ref, out_ref, t0_ref):
    cs = jax.nn.sigmoid(ctr_ref[...])
    s = jax.nn.sigmoid(cls_ref[...])
    sc = s * cs
    masked = jnp.where(sc > _CONF, sc, 0.0)
    out_ref[...] = masked

    rm = jnp.max(masked, axis=1, keepdims=True)
    lane = lax.broadcasted_iota(jnp.int32, (1, 128), 1).astype(jnp.float32)
    lo = jnp.float32(_CONF)
    width = (1.0 - _CONF)
    kth = jnp.float32(_K + 1)
    for _ in range(_REFINE):
        step = width / 128.0
        thr = lo + (lane + 1.0) * step
        cnt = jnp.sum((rm > thr).astype(jnp.float32), axis=0, keepdims=True)
        nup = jnp.sum((cnt >= kth).astype(jnp.float32))
        lo = lo + nup * step
        width = step
    t0_ref[0, 0] = lo


def _compact_body(scores_hbm, t0_hbm, vals_hbm, idxs_hbm,
                  buf_v, t0_v, cv_v, ci_v):
    wid = lax.axis_index("s") * 2 + lax.axis_index("c")
    base = wid * _SEG
    pltpu.sync_copy(scores_hbm.at[pl.ds(base, _SEG)], buf_v)
    pltpu.sync_copy(t0_hbm.at[pl.ds(0, 16)], t0_v)
    t0 = t0_v[...]
    zf = jnp.zeros((16,), jnp.float32)
    zi = jnp.zeros((16,), jnp.int32)
    for i in range(_CAP // 16):
        cv_v[pl.ds(i * 16, 16)] = zf
        ci_v[pl.ds(i * 16, 16)] = zi
    lanes = lax.iota(jnp.int32, 16)

    def step(k, cnt):
        v = buf_v[pl.ds(k * 16, 16)]
        m = v > t0
        inc = jnp.sum(m.astype(jnp.int32))

        @pl.when((inc > 0) & (cnt < _CAP - 16))
        def _():
            pos = cnt + plsc.cumsum(m.astype(jnp.int32)) - 1
            gidx = lanes + (base + k * 16)
            plsc.store_scatter(cv_v, [pos], v, mask=m)
            plsc.store_scatter(ci_v, [pos], gidx, mask=m)

        return cnt + jnp.where(cnt < _CAP - 16, inc, 0)

    lax.fori_loop(0, _SEG // 16, step, jnp.int32(0))
    pltpu.sync_copy(cv_v, vals_hbm.at[wid])
    pltpu.sync_copy(ci_v, idxs_hbm.at[wid])


@functools.cache
def _make_compact_kernel():
    return pl.kernel(
        _compact_body,
        out_type=[
            jax.ShapeDtypeStruct((_NW, _CAP), jnp.float32),
            jax.ShapeDtypeStruct((_NW, _CAP), jnp.int32),
        ],
        mesh=plsc.VectorSubcoreMesh(core_axis_name="c", subcore_axis_name="s"),
        compiler_params=pltpu.CompilerParams(needs_layout_passes=False),
        scratch_types=[
            pltpu.VMEM((_SEG,), jnp.float32),
            pltpu.VMEM((16,), jnp.float32),
            pltpu.VMEM((_CAP,), jnp.float32),
            pltpu.VMEM((_CAP,), jnp.int32),
        ],
    )


_SR = _NW * _CAP // 128


def _partner(x, d):
    if d < 128:
        lbit = lax.broadcasted_iota(jnp.int32, (_SR, 128), 1) & d
        lowv = pltpu.roll(x, 128 - d, 1)
        upv = pltpu.roll(x, d, 1)
        return jnp.where(lbit == 0, lowv, upv)
    dr = d // 128
    rbit = lax.broadcasted_iota(jnp.int32, (_SR, 128), 0) & dr
    lowv = pltpu.roll(x, _SR - dr, 0)
    upv = pltpu.roll(x, dr, 0)
    return jnp.where(rbit == 0, lowv, upv)


def _sort_body(vals_ref, idxs_ref, ts_ref, ti_ref):
    v = vals_ref[...]
    i = idxs_ref[...]
    fi = (lax.broadcasted_iota(jnp.int32, (_SR, 128), 0) * 128
          + lax.broadcasted_iota(jnp.int32, (_SR, 128), 1))
    nlev = (_SR * 128).bit_length() - 1
    for k in range(1, nlev + 1):
        up = ((fi >> k) & 1) == 0
        for j in reversed(range(k)):
            d = 1 << j
            pv = _partner(v, d)
            pi = _partner(i, d)
            lower = (fi & d) == 0
            self_first = (v > pv) | ((v == pv) & (i < pi))
            choose_self = (lower == up) == self_first
            v = jnp.where(choose_self, v, pv)
            i = jnp.where(choose_self, i, pi)
    ts_ref[...] = v[0:8, :]
    ti_ref[...] = i[0:8, :]


def _nms_body(ts_row_ref, ts_col_ref, ti_col_ref, ti_smem, bbox_ref,
              out_ref, boxes_ref, supu_ref, keep_ref):
    def gb(j, carry):
        r = ti_smem[j] // _C
        boxes_ref[pl.ds(j, 1), :] = bbox_ref[pl.ds(r, 1), :]
        return carry

    lax.fori_loop(0, _KP, gb, 0)

    lab_col = (ti_col_ref[...] % _C).astype(jnp.float32)
    offb = boxes_ref[...] + lab_col * _OFF

    ii = lax.broadcasted_iota(jnp.int32, (_KP, _KP), 0)
    jj = lax.broadcasted_iota(jnp.int32, (_KP, _KP), 1)
    ident = jnp.where(ii == jj, 1.0, 0.0)
    offr = lax.dot_general(offb, ident, (((0,), (0,)), ((), ())),
                           preferred_element_type=jnp.float32)

    x1c, y1c, x2c, y2c = (offb[:, k:k + 1] for k in range(4))
    x1r, y1r, x2r, y2r = (offr[k:k + 1, :] for k in range(4))
    area_c = jnp.maximum(x2c - x1c, 0.0) * jnp.maximum(y2c - y1c, 0.0)
    area_r = jnp.maximum(x2r - x1r, 0.0) * jnp.maximum(y2r - y1r, 0.0)
    ltx = jnp.maximum(x1c, x1r)
    lty = jnp.maximum(y1c, y1r)
    rbx = jnp.minimum(x2c, x2r)
    rby = jnp.minimum(y2c, y2r)
    inter = jnp.maximum(rbx - ltx, 0.0) * jnp.maximum(rby - lty, 0.0)
    union = area_c + area_r - inter
    iou = inter / jnp.maximum(union, 1e-9)
    supu_ref[...] = jnp.where((iou > _IOU) & (jj > ii), 1.0, 0.0)

    keep_ref[...] = jnp.where(ts_row_ref[...] > _CONF, 1.0, 0.0)
    lane = lax.broadcasted_iota(jnp.int32, (1, _KP), 1)

    def body(i, carry):
        row = supu_ref[pl.ds(i, 1), :]
        kv = keep_ref[...]
        ki = jnp.sum(jnp.where(lane == i, kv, 0.0))
        keep_ref[...] = kv * (1.0 - row * ki)
        return carry

    lax.fori_loop(0, _K, body, 0)

    keep_col = lax.dot_general(ident, keep_ref[...], (((1,), (1,)), ((), ())),
                               preferred_element_type=jnp.float32)

    out_ref[:, 0:4] = boxes_ref[0:_K, :]
    out_ref[:, 4:5] = ts_col_ref[0:_K, :]
    out_ref[:, 5:6] = keep_col[0:_K, :]


def kernel(cls_logits, bbox_reg, centerness, anchors):
    del anchors
    f32 = jnp.float32
    scores, t0 = pl.pallas_call(
        _scores_body,
        out_shape=[
            jax.ShapeDtypeStruct((_N, _C), f32),
            jax.ShapeDtypeStruct((1, 1), f32),
        ],
        out_specs=[
            pl.BlockSpec(memory_space=pltpu.VMEM),
            pl.BlockSpec(memory_space=pltpu.SMEM),
        ],
    )(cls_logits, centerness.reshape(_N, 1))

    t16 = jnp.broadcast_to(t0.reshape(1), (16,))
    vals, idxs = _make_compact_kernel()(scores.reshape(-1), t16)
    vals = vals.reshape(_SR, 128)
    idxs = idxs.reshape(_SR, 128)

    ts8, ti8 = pl.pallas_call(
        _sort_body,
        out_shape=[
            jax.ShapeDtypeStruct((8, 128), f32),
            jax.ShapeDtypeStruct((8, 128), jnp.int32),
        ],
    )(vals, idxs)

    ts_p = ts8.reshape(_KP)
    ti_p = ti8.reshape(_KP)

    out = pl.pallas_call(
        _nms_body,
        out_shape=jax.ShapeDtypeStruct((_K, 6), f32),
        in_specs=[
            pl.BlockSpec(memory_space=pltpu.VMEM),
            pl.BlockSpec(memory_space=pltpu.VMEM),
            pl.BlockSpec(memory_space=pltpu.VMEM),
            pl.BlockSpec(memory_space=pltpu.SMEM),
            pl.BlockSpec(memory_space=pltpu.VMEM),
        ],
        scratch_shapes=[
            pltpu.VMEM((_KP, 4), f32),
            pltpu.VMEM((_KP, _KP), f32),
            pltpu.VMEM((1, _KP), f32),
        ],
    )(ts_p.reshape(1, _KP), ts_p.reshape(_KP, 1), ti_p.reshape(_KP, 1),
      ti_p, bbox_reg)
    return out


# speedup vs baseline: 11.6733x; 1.0383x over previous
import functools

import jax
import jax.numpy as jnp
from jax import lax
from jax.experimental import pallas as pl
from jax.experimental.pallas import tpu as pltpu
from jax.experimental.pallas import tpu_sc as plsc

_N = 20000
_C = 80
_K = 1000
_KP = 1024
_CONF = 0.05
_IOU = 0.5
_OFF = 4096.0

_NW = 32
_SEG = _N * _C // _NW
_CAP = 256
_REFINE = 3


def _scores_body(cls_ref, ctr_ref, out_ref, t0_ref):
    cs = jax.nn.sigmoid(ctr_ref[...])
    s = jax.nn.sigmoid(cls_ref[...])
    sc = s * cs
    masked = jnp.where(sc > _CONF, sc, 0.0)
    out_ref[...] = masked

    rm = jnp.max(masked, axis=1, keepdims=True)
    lane = lax.broadcasted_iota(jnp.int32, (1, 128), 1).astype(jnp.float32)
    lo = jnp.float32(_CONF)
    width = (1.0 - _CONF)
    kth = jnp.float32(_K + 1)
    for _ in range(_REFINE):
        step = width / 128.0
        thr = lo + (lane + 1.0) * step
        cnt = jnp.sum((rm > thr).astype(jnp.float32), axis=0, keepdims=True)
        nup = jnp.sum((cnt >= kth).astype(jnp.float32))
        lo = lo + nup * step
        width = step
    t0_ref[0, 0] = lo


def _compact_body(scores_hbm, t0_hbm, vals_hbm, idxs_hbm,
                  buf_v, t0_v, cv_v, ci_v):
    wid = lax.axis_index("s") * 2 + lax.axis_index("c")
    base = wid * _SEG
    pltpu.sync_copy(scores_hbm.at[pl.ds(base, _SEG)], buf_v)
    pltpu.sync_copy(t0_hbm.at[pl.ds(0, 16)], t0_v)
    t0 = t0_v[...]
    zf = jnp.zeros((16,), jnp.float32)
    zi = jnp.zeros((16,), jnp.int32)
    for i in range(_CAP // 16):
        cv_v[pl.ds(i * 16, 16)] = zf
        ci_v[pl.ds(i * 16, 16)] = zi
    lanes = lax.iota(jnp.int32, 16)

    def step(k, cnt):
        v = buf_v[pl.ds(k * 16, 16)]
        m = v > t0
        inc = jnp.sum(m.astype(jnp.int32))

        @pl.when((inc > 0) & (cnt < _CAP - 16))
        def _():
            pos = cnt + plsc.cumsum(m.astype(jnp.int32)) - 1
            gidx = lanes + (base + k * 16)
            plsc.store_scatter(cv_v, [pos], v, mask=m)
            plsc.store_scatter(ci_v, [pos], gidx, mask=m)

        return cnt + jnp.where(cnt < _CAP - 16, inc, 0)

    lax.fori_loop(0, _SEG // 16, step, jnp.int32(0))
    pltpu.sync_copy(cv_v, vals_hbm.at[wid])
    pltpu.sync_copy(ci_v, idxs_hbm.at[wid])


@functools.cache
def _make_compact_kernel():
    return pl.kernel(
        _compact_body,
        out_type=[
            jax.ShapeDtypeStruct((_NW, _CAP), jnp.float32),
            jax.ShapeDtypeStruct((_NW, _CAP), jnp.int32),
        ],
        mesh=plsc.VectorSubcoreMesh(core_axis_name="c", subcore_axis_name="s"),
        compiler_params=pltpu.CompilerParams(needs_layout_passes=False),
        scratch_types=[
            pltpu.VMEM((_SEG,), jnp.float32),
            pltpu.VMEM((16,), jnp.float32),
            pltpu.VMEM((_CAP,), jnp.float32),
            pltpu.VMEM((_CAP,), jnp.int32),
        ],
    )


_SR = _NW * _CAP // 128


def _partner(x, d):
    if d < 128:
        lbit = lax.broadcasted_iota(jnp.int32, (_SR, 128), 1) & d
        lowv = pltpu.roll(x, 128 - d, 1)
        upv = pltpu.roll(x, d, 1)
        return jnp.where(lbit == 0, lowv, upv)
    dr = d // 128
    rbit = lax.broadcasted_iota(jnp.int32, (_SR, 128), 0) & dr
    lowv = pltpu.roll(x, _SR - dr, 0)
    upv = pltpu.roll(x, dr, 0)
    return jnp.where(rbit == 0, lowv, upv)


def _sort_body(vals_ref, idxs_ref, ts_ref, ti_ref):
    v = vals_ref[...]
    i = idxs_ref[...]
    fi = (lax.broadcasted_iota(jnp.int32, (_SR, 128), 0) * 128
          + lax.broadcasted_iota(jnp.int32, (_SR, 128), 1))
    nlev = (_SR * 128).bit_length() - 1
    for k in range(1, nlev + 1):
        up = ((fi >> k) & 1) == 0
        for j in reversed(range(k)):
            d = 1 << j
            pv = _partner(v, d)
            pi = _partner(i, d)
            lower = (fi & d) == 0
            self_first = (v > pv) | ((v == pv) & (i < pi))
            choose_self = (lower == up) == self_first
            v = jnp.where(choose_self, v, pv)
            i = jnp.where(choose_self, i, pi)
    ts_ref[...] = v[0:8, :]
    ti_ref[...] = i[0:8, :]


def _nms_body(ts_row_ref, ts_col_ref, ti_col_ref, ti_smem, bbox_ref,
              out_ref, boxes_ref, supu_ref, keep_ref):
    def gb(j, carry):
        r = ti_smem[j] // _C
        boxes_ref[pl.ds(j, 1), :] = bbox_ref[pl.ds(r, 1), :]
        return carry

    lax.fori_loop(0, _KP, gb, 0)

    lab_col = (ti_col_ref[...] % _C).astype(jnp.float32)
    offb = boxes_ref[...] + lab_col * _OFF

    ii = lax.broadcasted_iota(jnp.int32, (_KP, _KP), 0)
    jj = lax.broadcasted_iota(jnp.int32, (_KP, _KP), 1)
    ident = jnp.where(ii == jj, 1.0, 0.0)
    offr = lax.dot_general(offb, ident, (((0,), (0,)), ((), ())),
                           preferred_element_type=jnp.float32)

    x1c, y1c, x2c, y2c = (offb[:, k:k + 1] for k in range(4))
    x1r, y1r, x2r, y2r = (offr[k:k + 1, :] for k in range(4))
    area_c = jnp.maximum(x2c - x1c, 0.0) * jnp.maximum(y2c - y1c, 0.0)
    area_r = jnp.maximum(x2r - x1r, 0.0) * jnp.maximum(y2r - y1r, 0.0)
    ltx = jnp.maximum(x1c, x1r)
    lty = jnp.maximum(y1c, y1r)
    rbx = jnp.minimum(x2c, x2r)
    rby = jnp.minimum(y2c, y2r)
    inter = jnp.maximum(rbx - ltx, 0.0) * jnp.maximum(rby - lty, 0.0)
    union = area_c + area_r - inter
    iou = inter / jnp.maximum(union, 1e-9)
    supv = jnp.where((iou > _IOU) & (jj > ii), 1.0, 0.0)
    for t in range(_KP // 128):
        supu_ref[t] = supv[:, t * 128:(t + 1) * 128]

    keep_ref[...] = jnp.where(ts_row_ref[...] > _CONF, 1.0, 0.0)
    lane128 = lax.broadcasted_iota(jnp.int32, (1, 128), 1)

    for t in range(_KP // 128):
        c0 = t * 128
        keepc = keep_ref[0:1, c0:c0 + 128]
        if t > 0:
            prev = keep_ref[0:1, 0:c0]
            supb = supu_ref[t, 0:c0, :]
            pres = lax.dot_general(prev, supb, (((1,), (0,)), ((), ())),
                                   preferred_element_type=jnp.float32)
            keepc = keepc * jnp.where(pres > 0.0, 0.0, 1.0)

        nin = min(128, _K - c0)

        def body(i, kc, c0=c0, t=t):
            row = supu_ref[t, pl.ds(c0 + i, 1), :]
            ki = jnp.sum(jnp.where(lane128 == i, kc, 0.0))
            return kc * (1.0 - row * ki)

        keepc = lax.fori_loop(0, nin, body, keepc)
        keep_ref[0:1, c0:c0 + 128] = keepc

    keep_col = lax.dot_general(ident, keep_ref[...], (((1,), (1,)), ((), ())),
                               preferred_element_type=jnp.float32)

    out_ref[:, 0:4] = boxes_ref[0:_K, :]
    out_ref[:, 4:5] = ts_col_ref[0:_K, :]
    out_ref[:, 5:6] = keep_col[0:_K, :]


def kernel(cls_logits, bbox_reg, centerness, anchors):
    del anchors
    f32 = jnp.float32
    scores, t0 = pl.pallas_call(
        _scores_body,
        out_shape=[
            jax.ShapeDtypeStruct((_N, _C), f32),
            jax.ShapeDtypeStruct((1, 1), f32),
        ],
        out_specs=[
            pl.BlockSpec(memory_space=pltpu.VMEM),
            pl.BlockSpec(memory_space=pltpu.SMEM),
        ],
    )(cls_logits, centerness.reshape(_N, 1))

    t16 = jnp.broadcast_to(t0.reshape(1), (16,))
    vals, idxs = _make_compact_kernel()(scores.reshape(-1), t16)
    vals = vals.reshape(_SR, 128)
    idxs = idxs.reshape(_SR, 128)

    ts8, ti8 = pl.pallas_call(
        _sort_body,
        out_shape=[
            jax.ShapeDtypeStruct((8, 128), f32),
            jax.ShapeDtypeStruct((8, 128), jnp.int32),
        ],
    )(vals, idxs)

    ts_p = ts8.reshape(_KP)
    ti_p = ti8.reshape(_KP)

    out = pl.pallas_call(
        _nms_body,
        out_shape=jax.ShapeDtypeStruct((_K, 6), f32),
        in_specs=[
            pl.BlockSpec(memory_space=pltpu.VMEM),
            pl.BlockSpec(memory_space=pltpu.VMEM),
            pl.BlockSpec(memory_space=pltpu.VMEM),
            pl.BlockSpec(memory_space=pltpu.SMEM),
            pl.BlockSpec(memory_space=pltpu.VMEM),
        ],
        scratch_shapes=[
            pltpu.VMEM((_KP, 4), f32),
            pltpu.VMEM((_KP // 128, _KP, 128), f32),
            pltpu.VMEM((1, _KP), f32),
        ],
    )(ts_p.reshape(1, _KP), ts_p.reshape(_KP, 1), ti_p.reshape(_KP, 1),
      ti_p, bbox_reg)
    return out


# speedup vs baseline: 18.5184x; 1.5864x over previous
import functools

import jax
import jax.numpy as jnp
from jax import lax
from jax.experimental import pallas as pl
from jax.experimental.pallas import tpu as pltpu
from jax.experimental.pallas import tpu_sc as plsc

_N = 20000
_C = 80
_K = 1000
_KP = 1024
_CONF = 0.05
_IOU = 0.5
_OFF = 4096.0

_NW = 32
_SEG = _N * _C // _NW
_CAP = 256
_REFINE = 3


def _scores_body(cls_ref, ctr_ref, out_ref, t0_ref):
    cs = jax.nn.sigmoid(ctr_ref[...])
    s = jax.nn.sigmoid(cls_ref[...])
    sc = s * cs
    masked = jnp.where(sc > _CONF, sc, 0.0)
    out_ref[...] = masked

    rm = jnp.max(masked, axis=1, keepdims=True)
    lane = lax.broadcasted_iota(jnp.int32, (1, 128), 1).astype(jnp.float32)
    lo = jnp.float32(_CONF)
    width = (1.0 - _CONF)
    kth = jnp.float32(_K + 1)
    for _ in range(_REFINE):
        step = width / 128.0
        thr = lo + (lane + 1.0) * step
        cnt = jnp.sum((rm > thr).astype(jnp.float32), axis=0, keepdims=True)
        nup = jnp.sum((cnt >= kth).astype(jnp.float32))
        lo = lo + nup * step
        width = step
    t0_ref[0, 0] = lo


def _compact_body(scores_hbm, t0_hbm, vals_hbm, idxs_hbm,
                  buf_v, t0_v, cv_v, ci_v):
    wid = lax.axis_index("s") * 2 + lax.axis_index("c")
    base = wid * _SEG
    pltpu.sync_copy(scores_hbm.at[pl.ds(base, _SEG)], buf_v)
    pltpu.sync_copy(t0_hbm.at[pl.ds(0, 16)], t0_v)
    t0 = t0_v[...]
    zf = jnp.zeros((16,), jnp.float32)
    zi = jnp.zeros((16,), jnp.int32)
    for i in range(_CAP // 16):
        cv_v[pl.ds(i * 16, 16)] = zf
        ci_v[pl.ds(i * 16, 16)] = zi
    lanes = lax.iota(jnp.int32, 16)

    def step(k, cnt):
        v = buf_v[pl.ds(k * 16, 16)]
        m = v > t0
        inc = jnp.sum(m.astype(jnp.int32))

        @pl.when((inc > 0) & (cnt < _CAP - 16))
        def _():
            pos = cnt + plsc.cumsum(m.astype(jnp.int32)) - 1
            gidx = lanes + (base + k * 16)
            plsc.store_scatter(cv_v, [pos], v, mask=m)
            plsc.store_scatter(ci_v, [pos], gidx, mask=m)

        return cnt + jnp.where(cnt < _CAP - 16, inc, 0)

    lax.fori_loop(0, _SEG // 16, step, jnp.int32(0))
    pltpu.sync_copy(cv_v, vals_hbm.at[wid])
    pltpu.sync_copy(ci_v, idxs_hbm.at[wid])


@functools.cache
def _make_compact_kernel():
    return pl.kernel(
        _compact_body,
        out_type=[
            jax.ShapeDtypeStruct((_NW, _CAP), jnp.float32),
            jax.ShapeDtypeStruct((_NW, _CAP), jnp.int32),
        ],
        mesh=plsc.VectorSubcoreMesh(core_axis_name="c", subcore_axis_name="s"),
        compiler_params=pltpu.CompilerParams(needs_layout_passes=False),
        scratch_types=[
            pltpu.VMEM((_SEG,), jnp.float32),
            pltpu.VMEM((16,), jnp.float32),
            pltpu.VMEM((_CAP,), jnp.float32),
            pltpu.VMEM((_CAP,), jnp.int32),
        ],
    )


_SR = _NW * _CAP // 128


def _partner(x, d):
    if d < 128:
        lbit = lax.broadcasted_iota(jnp.int32, (_SR, 128), 1) & d
        lowv = pltpu.roll(x, 128 - d, 1)
        upv = pltpu.roll(x, d, 1)
        return jnp.where(lbit == 0, lowv, upv)
    dr = d // 128
    rbit = lax.broadcasted_iota(jnp.int32, (_SR, 128), 0) & dr
    lowv = pltpu.roll(x, _SR - dr, 0)
    upv = pltpu.roll(x, dr, 0)
    return jnp.where(rbit == 0, lowv, upv)


def _sort_body(vals_ref, idxs_ref, ts_ref, ti_ref):
    v = vals_ref[...]
    i = idxs_ref[...]
    fi = (lax.broadcasted_iota(jnp.int32, (_SR, 128), 0) * 128
          + lax.broadcasted_iota(jnp.int32, (_SR, 128), 1))
    nlev = (_SR * 128).bit_length() - 1
    for k in range(1, nlev + 1):
        up = ((fi >> k) & 1) == 0
        for j in reversed(range(k)):
            d = 1 << j
            pv = _partner(v, d)
            pi = _partner(i, d)
            lower = (fi & d) == 0
            self_first = (v > pv) | ((v == pv) & (i < pi))
            choose_self = (lower == up) == self_first
            v = jnp.where(choose_self, v, pv)
            i = jnp.where(choose_self, i, pi)
    ts_ref[...] = v[0:8, :]
    ti_ref[...] = i[0:8, :]


def _nms_body(ts_row_ref, ts_col_ref, ti_col_ref, ti_smem, bbox_ref,
              out_ref, boxes_ref, supu_ref, keep_ref):
    def gb(j, carry):
        r = ti_smem[j] // _C
        boxes_ref[pl.ds(j, 1), :] = bbox_ref[pl.ds(r, 1), :]
        return carry

    lax.fori_loop(0, _KP, gb, 0)

    lab_col = (ti_col_ref[...] % _C).astype(jnp.float32)
    offb = boxes_ref[...] + lab_col * _OFF

    ii = lax.broadcasted_iota(jnp.int32, (_KP, _KP), 0)
    jj = lax.broadcasted_iota(jnp.int32, (_KP, _KP), 1)
    ident = jnp.where(ii == jj, 1.0, 0.0)
    offr = lax.dot_general(offb, ident, (((0,), (0,)), ((), ())),
                           preferred_element_type=jnp.float32)

    x1c, y1c, x2c, y2c = (offb[:, k:k + 1] for k in range(4))
    x1r, y1r, x2r, y2r = (offr[k:k + 1, :] for k in range(4))
    area_c = jnp.maximum(x2c - x1c, 0.0) * jnp.maximum(y2c - y1c, 0.0)
    area_r = jnp.maximum(x2r - x1r, 0.0) * jnp.maximum(y2r - y1r, 0.0)
    ltx = jnp.maximum(x1c, x1r)
    lty = jnp.maximum(y1c, y1r)
    rbx = jnp.minimum(x2c, x2r)
    rby = jnp.minimum(y2c, y2r)
    inter = jnp.maximum(rbx - ltx, 0.0) * jnp.maximum(rby - lty, 0.0)
    union = area_c + area_r - inter
    iou = inter / jnp.maximum(union, 1e-9)
    supu_ref[...] = jnp.where((iou > _IOU) & (jj > ii), 1.0, 0.0)

    valid = jnp.where(ts_row_ref[...] > _CONF, 1.0, 0.0)
    kpad = jnp.where(lax.broadcasted_iota(jnp.int32, (1, _KP), 1) < _K,
                     1.0, 0.0)
    valid = valid * kpad

    def w_cond(c):
        return c[1]

    def w_body(c):
        keep, _ = c
        s = lax.dot_general(keep, supu_ref[...], (((1,), (0,)), ((), ())),
                            preferred_element_type=jnp.float32)
        nk = valid * jnp.where(s > 0.0, 0.0, 1.0)
        changed = jnp.sum(jnp.abs(nk - keep)) > 0.0
        return nk, changed

    keep, _ = lax.while_loop(w_cond, w_body, (valid, True))
    keep_ref[...] = keep

    keep_col = lax.dot_general(ident, keep_ref[...], (((1,), (1,)), ((), ())),
                               preferred_element_type=jnp.float32)

    out_ref[:, 0:4] = boxes_ref[0:_K, :]
    out_ref[:, 4:5] = ts_col_ref[0:_K, :]
    out_ref[:, 5:6] = keep_col[0:_K, :]


def kernel(cls_logits, bbox_reg, centerness, anchors):
    del anchors
    f32 = jnp.float32
    scores, t0 = pl.pallas_call(
        _scores_body,
        out_shape=[
            jax.ShapeDtypeStruct((_N, _C), f32),
            jax.ShapeDtypeStruct((1, 1), f32),
        ],
        out_specs=[
            pl.BlockSpec(memory_space=pltpu.VMEM),
            pl.BlockSpec(memory_space=pltpu.SMEM),
        ],
    )(cls_logits, centerness.reshape(_N, 1))

    t16 = jnp.broadcast_to(t0.reshape(1), (16,))
    vals, idxs = _make_compact_kernel()(scores.reshape(-1), t16)
    vals = vals.reshape(_SR, 128)
    idxs = idxs.reshape(_SR, 128)

    ts8, ti8 = pl.pallas_call(
        _sort_body,
        out_shape=[
            jax.ShapeDtypeStruct((8, 128), f32),
            jax.ShapeDtypeStruct((8, 128), jnp.int32),
        ],
    )(vals, idxs)

    ts_p = ts8.reshape(_KP)
    ti_p = ti8.reshape(_KP)

    out = pl.pallas_call(
        _nms_body,
        out_shape=jax.ShapeDtypeStruct((_K, 6), f32),
        in_specs=[
            pl.BlockSpec(memory_space=pltpu.VMEM),
            pl.BlockSpec(memory_space=pltpu.VMEM),
            pl.BlockSpec(memory_space=pltpu.VMEM),
            pl.BlockSpec(memory_space=pltpu.SMEM),
            pl.BlockSpec(memory_space=pltpu.VMEM),
        ],
        scratch_shapes=[
            pltpu.VMEM((_KP, 4), f32),
            pltpu.VMEM((_KP, _KP), f32),
            pltpu.VMEM((1, _KP), f32),
        ],
    )(ts_p.reshape(1, _KP), ts_p.reshape(_KP, 1), ti_p.reshape(_KP, 1),
      ti_p, bbox_reg)
    return out


# speedup vs baseline: 24.4908x; 1.3225x over previous
import functools

import jax
import jax.numpy as jnp
from jax import lax
from jax.experimental import pallas as pl
from jax.experimental.pallas import tpu as pltpu
from jax.experimental.pallas import tpu_sc as plsc

_N = 20000
_C = 80
_K = 1000
_KP = 1024
_CONF = 0.05
_IOU = 0.5
_OFF = 4096.0

_NW = 32
_SEG = _N * _C // _NW
_CAP = 256
_REFINE = 3


def _scores_body(cls_ref, ctr_ref, out_ref, t0_ref, rm_ref):
    cs = jax.nn.sigmoid(ctr_ref[...])
    s = jax.nn.sigmoid(cls_ref[...])
    sc = s * cs
    masked = jnp.where(sc > _CONF, sc, 0.0)
    out_ref[...] = masked

    rm = jnp.max(masked, axis=1, keepdims=True)
    rm_ref[...] = rm
    lane = lax.broadcasted_iota(jnp.int32, (1, 128), 1).astype(jnp.float32)
    lo = jnp.float32(_CONF)
    width = (1.0 - _CONF)
    kth = jnp.float32(_K + 1)
    for _ in range(_REFINE):
        step = width / 128.0
        thr = lo + (lane + 1.0) * step
        cnt = jnp.sum((rm > thr).astype(jnp.float32), axis=0, keepdims=True)
        nup = jnp.sum((cnt >= kth).astype(jnp.float32))
        lo = lo + nup * step
        width = step
    t0_ref[0, 0] = lo


_ROWS = _N // _NW
_RMP = 648


def _compact_body(scores_hbm, rm_hbm, t0_hbm, vals_hbm, idxs_hbm,
                  buf_v, rm_v, t0_v, cv_v, ci_v, cnt_ref):
    wid = lax.axis_index("s") * 2 + lax.axis_index("c")
    base = wid * _SEG
    pltpu.sync_copy(scores_hbm.at[pl.ds(base, _SEG)], buf_v)
    pltpu.sync_copy(rm_hbm.at[wid], rm_v)
    pltpu.sync_copy(t0_hbm.at[pl.ds(0, 16)], t0_v)
    t0 = t0_v[...]
    zf = jnp.zeros((16,), jnp.float32)
    zi16 = zf
    zi = jnp.zeros((16,), jnp.int32)
    for i in range(_CAP // 16):
        cv_v[pl.ds(i * 16, 16)] = zf
        ci_v[pl.ds(i * 16, 16)] = zi
    lanes = lax.iota(jnp.int32, 16)
    t0s = t0[0]

    def rstep(r, cnt):
        cnt_ref[0] = cnt
        rmr = rm_v[pl.ds(r, 16)][0]

        @pl.when(rmr > t0s)
        def _():
            rb = r * _C

            def gstep(g, cnt2):
                v = buf_v[pl.ds(rb + g * 16, 16)]
                m = v > t0
                inc = jnp.sum(m.astype(jnp.int32))

                @pl.when((inc > 0) & (cnt2 < _CAP - 16))
                def _():
                    pos = cnt2 + plsc.cumsum(m.astype(jnp.int32)) - 1
                    gidx = lanes + (base + rb + g * 16)
                    plsc.store_scatter(cv_v, [pos], v, mask=m)
                    plsc.store_scatter(ci_v, [pos], gidx, mask=m)

                return cnt2 + jnp.where(cnt2 < _CAP - 16, inc, 0)

            cnt_ref[0] = lax.fori_loop(0, _C // 16, gstep, cnt)

        return cnt_ref[0]

    lax.fori_loop(0, _ROWS, rstep, jnp.int32(0))
    pltpu.sync_copy(cv_v, vals_hbm.at[wid])
    pltpu.sync_copy(ci_v, idxs_hbm.at[wid])


@functools.cache
def _make_compact_kernel():
    return pl.kernel(
        _compact_body,
        out_type=[
            jax.ShapeDtypeStruct((_NW, _CAP), jnp.float32),
            jax.ShapeDtypeStruct((_NW, _CAP), jnp.int32),
        ],
        mesh=plsc.VectorSubcoreMesh(core_axis_name="c", subcore_axis_name="s"),
        compiler_params=pltpu.CompilerParams(needs_layout_passes=False),
        scratch_types=[
            pltpu.VMEM((_SEG,), jnp.float32),
            pltpu.VMEM((_RMP,), jnp.float32),
            pltpu.VMEM((16,), jnp.float32),
            pltpu.VMEM((_CAP,), jnp.float32),
            pltpu.VMEM((_CAP,), jnp.int32),
            pltpu.SMEM((1,), jnp.int32),
        ],
    )


_SR = _NW * _CAP // 128


def _partner(x, d):
    if d < 128:
        lbit = lax.broadcasted_iota(jnp.int32, (_SR, 128), 1) & d
        lowv = pltpu.roll(x, 128 - d, 1)
        upv = pltpu.roll(x, d, 1)
        return jnp.where(lbit == 0, lowv, upv)
    dr = d // 128
    rbit = lax.broadcasted_iota(jnp.int32, (_SR, 128), 0) & dr
    lowv = pltpu.roll(x, _SR - dr, 0)
    upv = pltpu.roll(x, dr, 0)
    return jnp.where(rbit == 0, lowv, upv)


def _sort_body(vals_ref, idxs_ref, ts_ref, ti_ref):
    v = vals_ref[...]
    i = idxs_ref[...]
    fi = (lax.broadcasted_iota(jnp.int32, (_SR, 128), 0) * 128
          + lax.broadcasted_iota(jnp.int32, (_SR, 128), 1))
    nlev = (_SR * 128).bit_length() - 1
    for k in range(1, nlev + 1):
        up = ((fi >> k) & 1) == 0
        for j in reversed(range(k)):
            d = 1 << j
            pv = _partner(v, d)
            pi = _partner(i, d)
            lower = (fi & d) == 0
            self_first = (v > pv) | ((v == pv) & (i < pi))
            choose_self = (lower == up) == self_first
            v = jnp.where(choose_self, v, pv)
            i = jnp.where(choose_self, i, pi)
    ts_ref[...] = v[0:8, :]
    ti_ref[...] = i[0:8, :]


def _nms_body(ts_row_ref, ts_col_ref, ti_col_ref, ti_smem, bbox_ref,
              out_ref, boxes_ref, supu_ref, keep_ref):
    def gb(j, carry):
        r = ti_smem[j] // _C
        boxes_ref[pl.ds(j, 1), :] = bbox_ref[pl.ds(r, 1), :]
        return carry

    lax.fori_loop(0, _KP, gb, 0)

    lab_col = (ti_col_ref[...] % _C).astype(jnp.float32)
    offb = boxes_ref[...] + lab_col * _OFF

    ii = lax.broadcasted_iota(jnp.int32, (_KP, _KP), 0)
    jj = lax.broadcasted_iota(jnp.int32, (_KP, _KP), 1)
    ident = jnp.where(ii == jj, 1.0, 0.0)
    offr = lax.dot_general(offb, ident, (((0,), (0,)), ((), ())),
                           preferred_element_type=jnp.float32)

    x1c, y1c, x2c, y2c = (offb[:, k:k + 1] for k in range(4))
    x1r, y1r, x2r, y2r = (offr[k:k + 1, :] for k in range(4))
    area_c = jnp.maximum(x2c - x1c, 0.0) * jnp.maximum(y2c - y1c, 0.0)
    area_r = jnp.maximum(x2r - x1r, 0.0) * jnp.maximum(y2r - y1r, 0.0)
    ltx = jnp.maximum(x1c, x1r)
    lty = jnp.maximum(y1c, y1r)
    rbx = jnp.minimum(x2c, x2r)
    rby = jnp.minimum(y2c, y2r)
    inter = jnp.maximum(rbx - ltx, 0.0) * jnp.maximum(rby - lty, 0.0)
    union = area_c + area_r - inter
    iou = inter / jnp.maximum(union, 1e-9)
    supu_ref[...] = jnp.where((iou > _IOU) & (jj > ii), 1.0, 0.0)

    valid = jnp.where(ts_row_ref[...] > _CONF, 1.0, 0.0)
    kpad = jnp.where(lax.broadcasted_iota(jnp.int32, (1, _KP), 1) < _K,
                     1.0, 0.0)
    valid = valid * kpad

    def w_cond(c):
        return c[1]

    def w_body(c):
        keep, _ = c
        s = lax.dot_general(keep, supu_ref[...], (((1,), (0,)), ((), ())),
                            preferred_element_type=jnp.float32)
        nk = valid * jnp.where(s > 0.0, 0.0, 1.0)
        changed = jnp.sum(jnp.abs(nk - keep)) > 0.0
        return nk, changed

    keep, _ = lax.while_loop(w_cond, w_body, (valid, True))
    keep_ref[...] = keep

    keep_col = lax.dot_general(ident, keep_ref[...], (((1,), (1,)), ((), ())),
                               preferred_element_type=jnp.float32)

    out_ref[:, 0:4] = boxes_ref[0:_K, :]
    out_ref[:, 4:5] = ts_col_ref[0:_K, :]
    out_ref[:, 5:6] = keep_col[0:_K, :]


def kernel(cls_logits, bbox_reg, centerness, anchors):
    del anchors
    f32 = jnp.float32
    scores, t0, rm = pl.pallas_call(
        _scores_body,
        out_shape=[
            jax.ShapeDtypeStruct((_N, _C), f32),
            jax.ShapeDtypeStruct((1, 1), f32),
            jax.ShapeDtypeStruct((_N, 1), f32),
        ],
        out_specs=[
            pl.BlockSpec(memory_space=pltpu.VMEM),
            pl.BlockSpec(memory_space=pltpu.SMEM),
            pl.BlockSpec(memory_space=pltpu.VMEM),
        ],
    )(cls_logits, centerness.reshape(_N, 1))

    t16 = jnp.broadcast_to(t0.reshape(1), (16,))
    rm2 = jnp.pad(rm.reshape(_NW, _ROWS), ((0, 0), (0, _RMP - _ROWS)))
    vals, idxs = _make_compact_kernel()(scores.reshape(-1), rm2, t16)
    vals = vals.reshape(_SR, 128)
    idxs = idxs.reshape(_SR, 128)

    ts8, ti8 = pl.pallas_call(
        _sort_body,
        out_shape=[
            jax.ShapeDtypeStruct((8, 128), f32),
            jax.ShapeDtypeStruct((8, 128), jnp.int32),
        ],
    )(vals, idxs)

    ts_p = ts8.reshape(_KP)
    ti_p = ti8.reshape(_KP)

    out = pl.pallas_call(
        _nms_body,
        out_shape=jax.ShapeDtypeStruct((_K, 6), f32),
        in_specs=[
            pl.BlockSpec(memory_space=pltpu.VMEM),
            pl.BlockSpec(memory_space=pltpu.VMEM),
            pl.BlockSpec(memory_space=pltpu.VMEM),
            pl.BlockSpec(memory_space=pltpu.SMEM),
            pl.BlockSpec(memory_space=pltpu.VMEM),
        ],
        scratch_shapes=[
            pltpu.VMEM((_KP, 4), f32),
            pltpu.VMEM((_KP, _KP), f32),
            pltpu.VMEM((1, _KP), f32),
        ],
    )(ts_p.reshape(1, _KP), ts_p.reshape(_KP, 1), ti_p.reshape(_KP, 1),
      ti_p, bbox_reg)
    return out
